# baseline (speedup 1.0000x reference)
import functools

import jax
import jax.numpy as jnp
from jax import lax
from jax.experimental import pallas as pl
from jax.experimental.pallas import tpu as pltpu
from jax.experimental.pallas import tpu_sc as plsc

_NC = 2
_NS = 16
_NW = _NC * _NS
_B = 16384
_D = 64
_DP = 128
_BPW = _B // _NW
_L = 16
_CHUNK = 128
_NCHUNK = _BPW // _CHUNK
_NBUF = 2


def _build():
    mesh = plsc.VectorSubcoreMesh(core_axis_name="c", subcore_axis_name="s")

    @functools.partial(
        pl.kernel,
        out_type=jax.ShapeDtypeStruct((_B,), jnp.float32),
        mesh=mesh,
        scratch_types=[
            pltpu.VMEM((_NCHUNK, _CHUNK), jnp.int32),
            pltpu.VMEM((_NCHUNK, _CHUNK), jnp.int32),
            pltpu.VMEM((_NBUF, _CHUNK, _DP), jnp.float32),
            pltpu.VMEM((_NBUF, _CHUNK, _DP), jnp.float32),
            pltpu.VMEM((_L, 17), jnp.float32),
            pltpu.VMEM((_BPW,), jnp.float32),
            pltpu.SemaphoreType.DMA,
            pltpu.SemaphoreType.DMA,
        ],
        compiler_params=pltpu.CompilerParams(needs_layout_passes=False),
    )
    def run(user_h, item_h, ut_h, it_h, out_h, uidx, iidx, ubuf, ibuf, smat,
            outv, sem0, sem1):
        sems = (sem0, sem1)
        wid = lax.axis_index("s") * _NC + lax.axis_index("c")
        base = wid * _BPW

        for j in range(_NCHUNK):
            pltpu.sync_copy(user_h.at[pl.ds(base + j * _CHUNK, _CHUNK)], uidx.at[j])
            pltpu.sync_copy(item_h.at[pl.ds(base + j * _CHUNK, _CHUNK)], iidx.at[j])

        def gather(j):
            slot = j % _NBUF
            cu = pltpu.async_copy(ut_h.at[uidx.at[j]], ubuf.at[slot], sems[slot])
            ci = pltpu.async_copy(it_h.at[iidx.at[j]], ibuf.at[slot], sems[slot])
            return cu, ci

        lanes = lax.iota(jnp.int32, _L)

        def compute(j, slot):
            for g in range(_CHUNK // _L):
                rbase = g * _L
                for r in range(_L):
                    s = None
                    for c in range(_D // _L):
                        u = ubuf[slot, rbase + r, pl.ds(c * _L, _L)]
                        v = ibuf[slot, rbase + r, pl.ds(c * _L, _L)]
                        s = u * v if s is None else s + u * v
                    smat[r, pl.ds(0, _L)] = s
                acc = jnp.zeros((_L,), jnp.float32)
                for k in range(_L):
                    col = plsc.load_gather(
                        smat, [lanes, jnp.full((_L,), k, jnp.int32)]
                    )
                    acc = acc + col
                outv[pl.ds(j * _CHUNK + rbase, _L)] = acc

        pending = gather(0)
        for j in range(_NCHUNK):
            nxt = gather(j + 1) if j + 1 < _NCHUNK else None
            cu, ci = pending
            cu.wait()
            ci.wait()
            compute(j, j % _NBUF)
            pending = nxt

        pltpu.sync_copy(outv, out_h.at[pl.ds(base, _BPW)])

    return run


_KERNEL = _build()


def kernel(user, item, user_table, item_table):
    ut = jnp.pad(user_table, ((0, 0), (0, _DP - _D)))
    it = jnp.pad(item_table, ((0, 0), (0, _DP - _D)))
    return _KERNEL(
        user.astype(jnp.int32),
        item.astype(jnp.int32),
        ut,
        it,
    )
